# baseline (speedup 1.0000x reference)
import jax
import jax.numpy as jnp
from jax import lax
from jax.experimental import pallas as pl
from jax.experimental.pallas import tpu as pltpu
from jax.experimental.pallas import tpu_sc as plsc

N_EMB = 8192
DIM = 32
ROWS = 8 * 1024
BLK = 256
N_BLK = ROWS // BLK

N_WORKERS = 32
B_PER_W = ROWS // N_WORKERS


CHUNK = 2048
N_CHUNK = N_EMB // CHUNK


def _dist_argmin_body(z_ref, w_ref, zn_ref, wn_ref, idx_ref, sse_ref):
    zb = z_ref[...]
    w = w_ref[...]
    a = zn_ref[...]
    b = wn_ref[...]
    m = lax.dot_general(zb, w, (((1,), (1,)), ((), ())),
                        preferred_element_type=jnp.float32)
    d = (a + b) - 2.0 * m

    col = lax.broadcasted_iota(jnp.int32, (BLK, CHUNK), 1)
    acc_v = None
    for k in range(N_CHUNK):
        dk = d[:, k * CHUNK:(k + 1) * CHUNK]
        cmin = jnp.min(dk, axis=1)
        ci = jnp.min(jnp.where(dk == cmin[:, None], col + k * CHUNK, N_EMB),
                     axis=1)
        if acc_v is None:
            acc_v = cmin.astype(jnp.bfloat16).astype(jnp.float32)
            acc_i = ci
            acc_e = cmin
        else:
            better = cmin < acc_v
            acc_v = jnp.where(better, cmin.astype(jnp.bfloat16).astype(jnp.float32), acc_v)
            acc_i = jnp.where(better, ci, acc_i)
            acc_e = jnp.where(better, cmin, acc_e)
    idx_ref[:, 0] = acc_i

    @pl.when(pl.program_id(0) == 0)
    def _():
        sse_ref[...] = jnp.zeros_like(sse_ref)

    sse_ref[...] += jnp.sum(acc_e) * (1.25 / float(ROWS * DIM))


IDX_CHUNK = 128
CHUNKS_PER_W = B_PER_W // IDX_CHUNK
PAD_D = 128


def _sc_gather_st_body(table_hbm, idx_hbm, z_hbm, out_hbm,
                       idx_v, rows_v, z_v, st_v, sem):
    wid = lax.axis_index("s") * 2 + lax.axis_index("c")
    base = wid * B_PER_W
    pltpu.sync_copy(idx_hbm.at[pl.ds(wid * CHUNKS_PER_W, CHUNKS_PER_W)], idx_v)
    copies = [
        pltpu.async_copy(table_hbm.at[idx_v.at[j]],
                         rows_v.at[pl.ds(j * IDX_CHUNK, IDX_CHUNK)], sem)
        for j in range(CHUNKS_PER_W)
    ]
    pltpu.sync_copy(z_hbm.at[pl.ds(base, B_PER_W)], z_v)
    for c in copies:
        c.wait()

    def body(r, carry):
        for c in (0, 16):
            v = rows_v[r, pl.ds(c, 16)]
            zz = z_v[r, pl.ds(c, 16)]
            st_v[r, pl.ds(c, 16)] = zz + (v - zz)
        return carry

    lax.fori_loop(0, B_PER_W, body, 0)
    pltpu.sync_copy(st_v, out_hbm.at[pl.ds(base, B_PER_W)])


def kernel(z, embedding_weight):
    z_flat = z.reshape(ROWS, DIM)
    z_bf16 = z_flat.astype(jnp.bfloat16)
    zn = jnp.sum(z_flat ** 2, axis=1, keepdims=True)
    wn = jnp.sum(embedding_weight ** 2, axis=1).reshape(1, N_EMB)

    idx2d, loss2d = pl.pallas_call(
        _dist_argmin_body,
        grid=(N_BLK,),
        in_specs=[
            pl.BlockSpec((BLK, DIM), lambda i: (i, 0)),
            pl.BlockSpec((N_EMB, DIM), lambda i: (0, 0)),
            pl.BlockSpec((BLK, 1), lambda i: (i, 0)),
            pl.BlockSpec((1, N_EMB), lambda i: (0, 0)),
        ],
        out_specs=[
            pl.BlockSpec((BLK, 1), lambda i: (i, 0)),
            pl.BlockSpec((1, 1), lambda i: (0, 0)),
        ],
        out_shape=[
            jax.ShapeDtypeStruct((ROWS, 1), jnp.int32),
            jax.ShapeDtypeStruct((1, 1), jnp.float32),
        ],
    )(z_bf16, embedding_weight, zn, wn)

    idx_flat = idx2d.reshape(ROWS)
    idx_chunks = idx2d.reshape(ROWS // IDX_CHUNK, IDX_CHUNK)
    table_pad = jnp.pad(embedding_weight, ((0, 0), (0, PAD_D - DIM)))

    sc_gather = pl.kernel(
        _sc_gather_st_body,
        out_type=jax.ShapeDtypeStruct((ROWS, DIM), jnp.float32),
        scratch_types=[
            pltpu.VMEM((CHUNKS_PER_W, IDX_CHUNK), jnp.int32),
            pltpu.VMEM((B_PER_W, PAD_D), jnp.float32),
            pltpu.VMEM((B_PER_W, DIM), jnp.float32),
            pltpu.VMEM((B_PER_W, DIM), jnp.float32),
            pltpu.SemaphoreType.DMA,
        ],
        mesh=plsc.VectorSubcoreMesh(core_axis_name="c", subcore_axis_name="s"),
    )
    z_q_st = sc_gather(table_pad, idx_chunks, z_flat)

    return (z_q_st.reshape(z.shape), loss2d[0, 0], idx_flat.reshape(z.shape[:-1]))


# speedup vs baseline: 1.1253x; 1.1253x over previous
import jax
import jax.numpy as jnp
from jax import lax
from jax.experimental import pallas as pl
from jax.experimental.pallas import tpu as pltpu
from jax.experimental.pallas import tpu_sc as plsc

N_EMB = 8192
DIM = 32
ROWS = 8 * 1024
BLK = 256
N_BLK = ROWS // BLK

N_WORKERS = 32
B_PER_W = ROWS // N_WORKERS


CHUNK = 2048
N_CHUNK = N_EMB // CHUNK

IDX_CHUNK = 128
CHUNKS_PER_W = B_PER_W // IDX_CHUNK
PAD_D = 128


def _dist_argmin_body(z_ref, w_ref, zn_ref, wn_ref, idx_ref, sse_ref, wpad_ref):
    zb = z_ref[...].astype(jnp.bfloat16)
    w = w_ref[...]
    a = zn_ref[...]
    b = wn_ref[...]
    m2 = lax.dot_general(zb, w * (-2.0), (((1,), (1,)), ((), ())),
                         preferred_element_type=jnp.float32)
    d = (a + b) + m2

    col = lax.broadcasted_iota(jnp.int32, (BLK, CHUNK), 1).astype(jnp.float32)
    acc_v = None
    for k in range(N_CHUNK):
        dk = d[:, k * CHUNK:(k + 1) * CHUNK]
        cmin = jnp.min(dk, axis=1)
        ci = jnp.min(jnp.where(dk == cmin[:, None], col, float(N_EMB)),
                     axis=1) + float(k * CHUNK)
        if acc_v is None:
            acc_v = cmin.astype(jnp.bfloat16).astype(jnp.float32)
            acc_i = ci
            acc_e = cmin
        else:
            better = cmin < acc_v
            acc_v = jnp.where(better, cmin.astype(jnp.bfloat16).astype(jnp.float32), acc_v)
            acc_i = jnp.where(better, ci, acc_i)
            acc_e = jnp.where(better, cmin, acc_e)
    idx_ref[:, 0] = acc_i.astype(jnp.int32)

    @pl.when(pl.program_id(0) == 0)
    def _():
        sse_ref[...] = jnp.zeros_like(sse_ref)
        wpad_ref[...] = jnp.pad(w, ((0, 0), (0, PAD_D - DIM)))

    sse_ref[...] += jnp.sum(acc_e) * (1.25 / float(ROWS * DIM))


def _sc_gather_st_body(table_hbm, idx_hbm, z_hbm, out_hbm,
                       idx_v, rows_v, z_v, st_v, sem):
    wid = lax.axis_index("s") * 2 + lax.axis_index("c")
    base = wid * B_PER_W
    pltpu.sync_copy(idx_hbm.at[pl.ds(wid * CHUNKS_PER_W, CHUNKS_PER_W)], idx_v)
    copies = [
        pltpu.async_copy(table_hbm.at[idx_v.at[j]],
                         rows_v.at[pl.ds(j * IDX_CHUNK, IDX_CHUNK)], sem)
        for j in range(CHUNKS_PER_W)
    ]
    pltpu.sync_copy(z_hbm.at[pl.ds(base, B_PER_W)], z_v)
    for c in copies:
        c.wait()

    def body(r, carry):
        for c in (0, 16):
            v = rows_v[r, pl.ds(c, 16)]
            zz = z_v[r, pl.ds(c, 16)]
            st_v[r, pl.ds(c, 16)] = zz + (v - zz)
        return carry

    lax.fori_loop(0, B_PER_W, body, 0)
    pltpu.sync_copy(st_v, out_hbm.at[pl.ds(base, B_PER_W)])


def kernel(z, embedding_weight):
    z_flat = z.reshape(ROWS, DIM)
    zn = jnp.sum(z_flat ** 2, axis=1, keepdims=True)
    wn = jnp.sum(embedding_weight ** 2, axis=1).reshape(1, N_EMB)

    idx2d, loss2d, table_pad = pl.pallas_call(
        _dist_argmin_body,
        grid=(N_BLK,),
        in_specs=[
            pl.BlockSpec((BLK, DIM), lambda i: (i, 0)),
            pl.BlockSpec((N_EMB, DIM), lambda i: (0, 0)),
            pl.BlockSpec((BLK, 1), lambda i: (i, 0)),
            pl.BlockSpec((1, N_EMB), lambda i: (0, 0)),
        ],
        out_specs=[
            pl.BlockSpec((BLK, 1), lambda i: (i, 0)),
            pl.BlockSpec((1, 1), lambda i: (0, 0)),
            pl.BlockSpec((N_EMB, PAD_D), lambda i: (0, 0)),
        ],
        out_shape=[
            jax.ShapeDtypeStruct((ROWS, 1), jnp.int32),
            jax.ShapeDtypeStruct((1, 1), jnp.float32),
            jax.ShapeDtypeStruct((N_EMB, PAD_D), jnp.float32),
        ],
    )(z_flat, embedding_weight, zn, wn)

    idx_flat = idx2d.reshape(ROWS)
    idx_chunks = idx2d.reshape(ROWS // IDX_CHUNK, IDX_CHUNK)

    sc_gather = pl.kernel(
        _sc_gather_st_body,
        out_type=jax.ShapeDtypeStruct((ROWS, DIM), jnp.float32),
        scratch_types=[
            pltpu.VMEM((CHUNKS_PER_W, IDX_CHUNK), jnp.int32),
            pltpu.VMEM((B_PER_W, PAD_D), jnp.float32),
            pltpu.VMEM((B_PER_W, DIM), jnp.float32),
            pltpu.VMEM((B_PER_W, DIM), jnp.float32),
            pltpu.SemaphoreType.DMA,
        ],
        mesh=plsc.VectorSubcoreMesh(core_axis_name="c", subcore_axis_name="s"),
    )
    z_q_st = sc_gather(table_pad, idx_chunks, z_flat)

    return (z_q_st.reshape(z.shape), loss2d[0, 0], idx_flat.reshape(z.shape[:-1]))


# speedup vs baseline: 1.1793x; 1.0480x over previous
import jax
import jax.numpy as jnp
from jax import lax
from jax.experimental import pallas as pl
from jax.experimental.pallas import tpu as pltpu
from jax.experimental.pallas import tpu_sc as plsc

N_EMB = 8192
DIM = 32
ROWS = 8 * 1024
BLK = 512
N_BLK = ROWS // BLK

N_WORKERS = 32
B_PER_W = ROWS // N_WORKERS


CHUNK = 2048
N_CHUNK = N_EMB // CHUNK

IDX_CHUNK = 128
CHUNKS_PER_W = B_PER_W // IDX_CHUNK
PAD_D = 128


def _dist_argmin_body(z_ref, w_ref, zn_ref, wn_ref, idx_ref, sse_ref, wpad_ref):
    zb = z_ref[...].astype(jnp.bfloat16)
    w = w_ref[...]
    a = zn_ref[...]
    b = wn_ref[...]
    m2 = lax.dot_general(zb, w * (-2.0), (((1,), (1,)), ((), ())),
                         preferred_element_type=jnp.float32)
    d = (a + b) + m2

    col = lax.broadcasted_iota(jnp.int32, (BLK, CHUNK), 1).astype(jnp.float32)
    acc_v = None
    for k in range(N_CHUNK):
        dk = d[:, k * CHUNK:(k + 1) * CHUNK]
        cmin = jnp.min(dk, axis=1)
        ci = jnp.min(jnp.where(dk == cmin[:, None], col, float(N_EMB)),
                     axis=1) + float(k * CHUNK)
        if acc_v is None:
            acc_v = cmin.astype(jnp.bfloat16).astype(jnp.float32)
            acc_i = ci
            acc_e = cmin
        else:
            better = cmin < acc_v
            acc_v = jnp.where(better, cmin.astype(jnp.bfloat16).astype(jnp.float32), acc_v)
            acc_i = jnp.where(better, ci, acc_i)
            acc_e = jnp.where(better, cmin, acc_e)
    idx_ref[:, 0] = acc_i.astype(jnp.int32)

    @pl.when(pl.program_id(0) == 0)
    def _():
        sse_ref[...] = jnp.zeros_like(sse_ref)
        wpad_ref[...] = jnp.pad(w, ((0, 0), (0, PAD_D - DIM)))

    sse_ref[...] += jnp.sum(acc_e) * (1.25 / float(ROWS * DIM))


def _sc_gather_st_body(table_hbm, idx_hbm, z_hbm, out_hbm,
                       idx_v, rows_v, z_v, st_v, sem):
    wid = lax.axis_index("s") * 2 + lax.axis_index("c")
    base = wid * B_PER_W
    pltpu.sync_copy(idx_hbm.at[pl.ds(wid * CHUNKS_PER_W, CHUNKS_PER_W)], idx_v)
    copies = [
        pltpu.async_copy(table_hbm.at[idx_v.at[j]],
                         rows_v.at[pl.ds(j * IDX_CHUNK, IDX_CHUNK)], sem)
        for j in range(CHUNKS_PER_W)
    ]
    pltpu.sync_copy(z_hbm.at[pl.ds(base, B_PER_W)], z_v)
    for c in copies:
        c.wait()

    def body(r, carry):
        for c in (0, 16):
            v = rows_v[r, pl.ds(c, 16)]
            zz = z_v[r, pl.ds(c, 16)]
            st_v[r, pl.ds(c, 16)] = zz + (v - zz)
        return carry

    lax.fori_loop(0, B_PER_W, body, 0)
    pltpu.sync_copy(st_v, out_hbm.at[pl.ds(base, B_PER_W)])


def kernel(z, embedding_weight):
    z_flat = z.reshape(ROWS, DIM)
    zn = jnp.sum(z_flat ** 2, axis=1, keepdims=True)
    wn = jnp.sum(embedding_weight ** 2, axis=1).reshape(1, N_EMB)

    idx2d, loss2d, table_pad = pl.pallas_call(
        _dist_argmin_body,
        grid=(N_BLK,),
        in_specs=[
            pl.BlockSpec((BLK, DIM), lambda i: (i, 0)),
            pl.BlockSpec((N_EMB, DIM), lambda i: (0, 0)),
            pl.BlockSpec((BLK, 1), lambda i: (i, 0)),
            pl.BlockSpec((1, N_EMB), lambda i: (0, 0)),
        ],
        out_specs=[
            pl.BlockSpec((BLK, 1), lambda i: (i, 0)),
            pl.BlockSpec((1, 1), lambda i: (0, 0)),
            pl.BlockSpec((N_EMB, PAD_D), lambda i: (0, 0)),
        ],
        out_shape=[
            jax.ShapeDtypeStruct((ROWS, 1), jnp.int32),
            jax.ShapeDtypeStruct((1, 1), jnp.float32),
            jax.ShapeDtypeStruct((N_EMB, PAD_D), jnp.float32),
        ],
    )(z_flat, embedding_weight, zn, wn)

    idx_flat = idx2d.reshape(ROWS)
    idx_chunks = idx2d.reshape(ROWS // IDX_CHUNK, IDX_CHUNK)

    sc_gather = pl.kernel(
        _sc_gather_st_body,
        out_type=jax.ShapeDtypeStruct((ROWS, DIM), jnp.float32),
        scratch_types=[
            pltpu.VMEM((CHUNKS_PER_W, IDX_CHUNK), jnp.int32),
            pltpu.VMEM((B_PER_W, PAD_D), jnp.float32),
            pltpu.VMEM((B_PER_W, DIM), jnp.float32),
            pltpu.VMEM((B_PER_W, DIM), jnp.float32),
            pltpu.SemaphoreType.DMA,
        ],
        mesh=plsc.VectorSubcoreMesh(core_axis_name="c", subcore_axis_name="s"),
    )
    z_q_st = sc_gather(table_pad, idx_chunks, z_flat)

    return (z_q_st.reshape(z.shape), loss2d[0, 0], idx_flat.reshape(z.shape[:-1]))


# speedup vs baseline: 1.2022x; 1.0194x over previous
import jax
import jax.numpy as jnp
from jax import lax
from jax.experimental import pallas as pl
from jax.experimental.pallas import tpu as pltpu
from jax.experimental.pallas import tpu_sc as plsc

N_EMB = 8192
DIM = 32
ROWS = 8 * 1024
BLK = 1024
N_BLK = ROWS // BLK

N_WORKERS = 32
B_PER_W = ROWS // N_WORKERS


CHUNK = 2048
N_CHUNK = N_EMB // CHUNK

IDX_CHUNK = 128
CHUNKS_PER_W = B_PER_W // IDX_CHUNK
PAD_D = 128


def _dist_argmin_body(z_ref, w_ref, zn_ref, wn_ref, idx_ref, sse_ref, wpad_ref):
    zb = z_ref[...].astype(jnp.bfloat16)
    w = w_ref[...]
    a = zn_ref[...]
    b = wn_ref[...]
    m2 = lax.dot_general(zb, w * (-2.0), (((1,), (1,)), ((), ())),
                         preferred_element_type=jnp.float32)
    d = (a + b) + m2

    col = lax.broadcasted_iota(jnp.int32, (BLK, CHUNK), 1).astype(jnp.float32)
    acc_v = None
    for k in range(N_CHUNK):
        dk = d[:, k * CHUNK:(k + 1) * CHUNK]
        cmin = jnp.min(dk, axis=1)
        ci = jnp.min(jnp.where(dk == cmin[:, None], col, float(N_EMB)),
                     axis=1) + float(k * CHUNK)
        if acc_v is None:
            acc_v = cmin.astype(jnp.bfloat16).astype(jnp.float32)
            acc_i = ci
            acc_e = cmin
        else:
            better = cmin < acc_v
            acc_v = jnp.where(better, cmin.astype(jnp.bfloat16).astype(jnp.float32), acc_v)
            acc_i = jnp.where(better, ci, acc_i)
            acc_e = jnp.where(better, cmin, acc_e)
    idx_ref[:, 0] = acc_i.astype(jnp.int32)

    @pl.when(pl.program_id(0) == 0)
    def _():
        sse_ref[...] = jnp.zeros_like(sse_ref)
        wpad_ref[...] = jnp.pad(w, ((0, 0), (0, PAD_D - DIM)))

    sse_ref[...] += jnp.sum(acc_e) * (1.25 / float(ROWS * DIM))


def _sc_gather_st_body(table_hbm, idx_hbm, z_hbm, out_hbm,
                       idx_v, rows_v, z_v, st_v, sem):
    wid = lax.axis_index("s") * 2 + lax.axis_index("c")
    base = wid * B_PER_W
    pltpu.sync_copy(idx_hbm.at[pl.ds(wid * CHUNKS_PER_W, CHUNKS_PER_W)], idx_v)
    copies = [
        pltpu.async_copy(table_hbm.at[idx_v.at[j]],
                         rows_v.at[pl.ds(j * IDX_CHUNK, IDX_CHUNK)], sem)
        for j in range(CHUNKS_PER_W)
    ]
    pltpu.sync_copy(z_hbm.at[pl.ds(base, B_PER_W)], z_v)
    for c in copies:
        c.wait()

    def body(r, carry):
        for c in (0, 16):
            v = rows_v[r, pl.ds(c, 16)]
            zz = z_v[r, pl.ds(c, 16)]
            st_v[r, pl.ds(c, 16)] = zz + (v - zz)
        return carry

    lax.fori_loop(0, B_PER_W, body, 0)
    pltpu.sync_copy(st_v, out_hbm.at[pl.ds(base, B_PER_W)])


def kernel(z, embedding_weight):
    z_flat = z.reshape(ROWS, DIM)
    zn = jnp.sum(z_flat ** 2, axis=1, keepdims=True)
    wn = jnp.sum(embedding_weight ** 2, axis=1).reshape(1, N_EMB)

    idx2d, loss2d, table_pad = pl.pallas_call(
        _dist_argmin_body,
        grid=(N_BLK,),
        in_specs=[
            pl.BlockSpec((BLK, DIM), lambda i: (i, 0)),
            pl.BlockSpec((N_EMB, DIM), lambda i: (0, 0)),
            pl.BlockSpec((BLK, 1), lambda i: (i, 0)),
            pl.BlockSpec((1, N_EMB), lambda i: (0, 0)),
        ],
        out_specs=[
            pl.BlockSpec((BLK, 1), lambda i: (i, 0)),
            pl.BlockSpec((1, 1), lambda i: (0, 0)),
            pl.BlockSpec((N_EMB, PAD_D), lambda i: (0, 0)),
        ],
        out_shape=[
            jax.ShapeDtypeStruct((ROWS, 1), jnp.int32),
            jax.ShapeDtypeStruct((1, 1), jnp.float32),
            jax.ShapeDtypeStruct((N_EMB, PAD_D), jnp.float32),
        ],
    )(z_flat, embedding_weight, zn, wn)

    idx_flat = idx2d.reshape(ROWS)
    idx_chunks = idx2d.reshape(ROWS // IDX_CHUNK, IDX_CHUNK)

    sc_gather = pl.kernel(
        _sc_gather_st_body,
        out_type=jax.ShapeDtypeStruct((ROWS, DIM), jnp.float32),
        scratch_types=[
            pltpu.VMEM((CHUNKS_PER_W, IDX_CHUNK), jnp.int32),
            pltpu.VMEM((B_PER_W, PAD_D), jnp.float32),
            pltpu.VMEM((B_PER_W, DIM), jnp.float32),
            pltpu.VMEM((B_PER_W, DIM), jnp.float32),
            pltpu.SemaphoreType.DMA,
        ],
        mesh=plsc.VectorSubcoreMesh(core_axis_name="c", subcore_axis_name="s"),
    )
    z_q_st = sc_gather(table_pad, idx_chunks, z_flat)

    return (z_q_st.reshape(z.shape), loss2d[0, 0], idx_flat.reshape(z.shape[:-1]))


# speedup vs baseline: 1.2391x; 1.0307x over previous
import jax
import jax.numpy as jnp
from jax import lax
from jax.experimental import pallas as pl
from jax.experimental.pallas import tpu as pltpu
from jax.experimental.pallas import tpu_sc as plsc

N_EMB = 8192
DIM = 32
ROWS = 8 * 1024
BLK = 1024
N_BLK = ROWS // BLK

N_WORKERS = 32
B_PER_W = ROWS // N_WORKERS


CHUNK = 2048
N_CHUNK = N_EMB // CHUNK

IDX_CHUNK = 128
CHUNKS_PER_W = B_PER_W // IDX_CHUNK
PAD_D = 128


def _dist_argmin_body(z_ref, w_ref, zn_ref, wn_ref, idx_ref, sse_ref, wpad_ref):
    zb = z_ref[...].astype(jnp.bfloat16)
    w = w_ref[...]
    a = zn_ref[...]
    b = wn_ref[...]
    m2 = lax.dot_general(zb, w * (-2.0), (((1,), (1,)), ((), ())),
                         preferred_element_type=jnp.float32)
    d = (a + b) + m2

    col = lax.broadcasted_iota(jnp.int32, (BLK, CHUNK), 1).astype(jnp.float32)
    acc_v = None
    for k in range(N_CHUNK):
        dk = d[:, k * CHUNK:(k + 1) * CHUNK]
        cmin = jnp.min(dk, axis=1)
        ci = jnp.min(jnp.where(dk == cmin[:, None], col, float(N_EMB)),
                     axis=1) + float(k * CHUNK)
        if acc_v is None:
            acc_v = cmin.astype(jnp.bfloat16).astype(jnp.float32)
            acc_i = ci
            acc_e = cmin
        else:
            better = cmin < acc_v
            acc_v = jnp.where(better, cmin.astype(jnp.bfloat16).astype(jnp.float32), acc_v)
            acc_i = jnp.where(better, ci, acc_i)
            acc_e = jnp.where(better, cmin, acc_e)
    idx_ref[...] = acc_i.astype(jnp.int32).reshape(BLK // 128, 128)

    @pl.when(pl.program_id(0) == 0)
    def _():
        sse_ref[...] = jnp.zeros_like(sse_ref)
        wpad_ref[...] = jnp.pad(w, ((0, 0), (0, PAD_D - DIM)))

    sse_ref[...] += jnp.sum(acc_e) * (1.25 / float(ROWS * DIM))


def _sc_gather_st_body(table_hbm, idx_hbm, z_hbm, out_hbm,
                       idx_v, rows_v, z_v, st_v, sem):
    wid = lax.axis_index("s") * 2 + lax.axis_index("c")
    base = wid * B_PER_W
    pltpu.sync_copy(idx_hbm.at[pl.ds(wid * CHUNKS_PER_W, CHUNKS_PER_W)], idx_v)
    copies = [
        pltpu.async_copy(table_hbm.at[idx_v.at[j]],
                         rows_v.at[pl.ds(j * IDX_CHUNK, IDX_CHUNK)], sem)
        for j in range(CHUNKS_PER_W)
    ]
    pltpu.sync_copy(z_hbm.at[pl.ds(base, B_PER_W)], z_v)
    for c in copies:
        c.wait()

    def body(r, carry):
        for c in (0, 16):
            v = rows_v[r, pl.ds(c, 16)]
            zz = z_v[r, pl.ds(c, 16)]
            st_v[r, pl.ds(c, 16)] = zz + (v - zz)
        return carry

    lax.fori_loop(0, B_PER_W, body, 0)
    pltpu.sync_copy(st_v, out_hbm.at[pl.ds(base, B_PER_W)])


def kernel(z, embedding_weight):
    z_flat = z.reshape(ROWS, DIM)
    zn = jnp.sum(z_flat ** 2, axis=1, keepdims=True)
    wn = jnp.sum(embedding_weight ** 2, axis=1).reshape(1, N_EMB)

    idx2d, loss2d, table_pad = pl.pallas_call(
        _dist_argmin_body,
        grid=(N_BLK,),
        in_specs=[
            pl.BlockSpec((BLK, DIM), lambda i: (i, 0)),
            pl.BlockSpec((N_EMB, DIM), lambda i: (0, 0)),
            pl.BlockSpec((BLK, 1), lambda i: (i, 0)),
            pl.BlockSpec((1, N_EMB), lambda i: (0, 0)),
        ],
        out_specs=[
            pl.BlockSpec((BLK // 128, 128), lambda i: (i, 0)),
            pl.BlockSpec((1, 1), lambda i: (0, 0)),
            pl.BlockSpec((N_EMB, PAD_D), lambda i: (0, 0)),
        ],
        out_shape=[
            jax.ShapeDtypeStruct((ROWS // 128, 128), jnp.int32),
            jax.ShapeDtypeStruct((1, 1), jnp.float32),
            jax.ShapeDtypeStruct((N_EMB, PAD_D), jnp.float32),
        ],
    )(z_flat, embedding_weight, zn, wn)

    idx_flat = idx2d.reshape(ROWS)
    idx_chunks = idx2d

    sc_gather = pl.kernel(
        _sc_gather_st_body,
        out_type=jax.ShapeDtypeStruct((ROWS, DIM), jnp.float32),
        scratch_types=[
            pltpu.VMEM((CHUNKS_PER_W, IDX_CHUNK), jnp.int32),
            pltpu.VMEM((B_PER_W, PAD_D), jnp.float32),
            pltpu.VMEM((B_PER_W, DIM), jnp.float32),
            pltpu.VMEM((B_PER_W, DIM), jnp.float32),
            pltpu.SemaphoreType.DMA,
        ],
        mesh=plsc.VectorSubcoreMesh(core_axis_name="c", subcore_axis_name="s"),
    )
    z_q_st = sc_gather(table_pad, idx_chunks, z_flat)

    return (z_q_st.reshape(z.shape), loss2d[0, 0], idx_flat.reshape(z.shape[:-1]))


# speedup vs baseline: 1.2418x; 1.0022x over previous
import jax
import jax.numpy as jnp
from jax import lax
from jax.experimental import pallas as pl
from jax.experimental.pallas import tpu as pltpu
from jax.experimental.pallas import tpu_sc as plsc

N_EMB = 8192
DIM = 32
ROWS = 8 * 1024
BLK = 1024
N_BLK = ROWS // BLK

N_WORKERS = 32
B_PER_W = ROWS // N_WORKERS


CHUNK = 2048
N_CHUNK = N_EMB // CHUNK

IDX_CHUNK = 128
CHUNKS_PER_W = B_PER_W // IDX_CHUNK
PAD_D = 128


def _dist_argmin_body(z_ref, w_ref, zn_ref, wn_ref, idx_ref, sse_ref, wpad_ref):
    zb = z_ref[...].astype(jnp.bfloat16)
    w = w_ref[...]
    a = zn_ref[...]
    b = wn_ref[...]
    m2 = lax.dot_general(zb, w * (-2.0), (((1,), (1,)), ((), ())),
                         preferred_element_type=jnp.float32)
    d = (a + b) + m2

    col = lax.broadcasted_iota(jnp.int32, (BLK, CHUNK), 1).astype(jnp.float32)
    acc_v = None
    for k in range(N_CHUNK):
        dk = d[:, k * CHUNK:(k + 1) * CHUNK]
        cmin = jnp.min(dk, axis=1)
        ci = jnp.min(jnp.where(dk == cmin[:, None], col, float(N_EMB)),
                     axis=1) + float(k * CHUNK)
        if acc_v is None:
            acc_v = cmin.astype(jnp.bfloat16).astype(jnp.float32)
            acc_i = ci
            acc_e = cmin
        else:
            better = cmin < acc_v
            acc_v = jnp.where(better, cmin.astype(jnp.bfloat16).astype(jnp.float32), acc_v)
            acc_i = jnp.where(better, ci, acc_i)
            acc_e = jnp.where(better, cmin, acc_e)
    idx_ref[...] = acc_i.astype(jnp.int32).reshape(BLK // 128, 128)

    @pl.when(pl.program_id(0) == 0)
    def _():
        sse_ref[...] = jnp.zeros_like(sse_ref)
        wpad_ref[...] = jnp.pad(w, ((0, 0), (0, PAD_D - DIM)))

    sse_ref[...] += jnp.sum(acc_e) * (1.25 / float(ROWS * DIM))


def _sc_gather_st_body(table_hbm, idx_hbm, z_hbm, out_hbm,
                       idx_v, rows_v, z_v, st_v, sem):
    wid = lax.axis_index("s") * 2 + lax.axis_index("c")
    base = wid * B_PER_W
    pltpu.sync_copy(idx_hbm.at[pl.ds(wid * CHUNKS_PER_W, CHUNKS_PER_W)], idx_v)
    copies = [
        pltpu.async_copy(table_hbm.at[idx_v.at[j]],
                         rows_v.at[pl.ds(j * IDX_CHUNK, IDX_CHUNK)], sem)
        for j in range(CHUNKS_PER_W)
    ]
    pltpu.sync_copy(z_hbm.at[pl.ds(base, B_PER_W)], z_v)
    for c in copies:
        c.wait()

    def body(r4, carry):
        for u in range(4):
            r = r4 * 4 + u
            for c in (0, 16):
                v = rows_v[r, pl.ds(c, 16)]
                zz = z_v[r, pl.ds(c, 16)]
                st_v[r, pl.ds(c, 16)] = zz + (v - zz)
        return carry

    lax.fori_loop(0, B_PER_W // 4, body, 0)
    pltpu.sync_copy(st_v, out_hbm.at[pl.ds(base, B_PER_W)])


def kernel(z, embedding_weight):
    z_flat = z.reshape(ROWS, DIM)
    zn = jnp.sum(z_flat ** 2, axis=1, keepdims=True)
    wn = jnp.sum(embedding_weight ** 2, axis=1).reshape(1, N_EMB)

    idx2d, loss2d, table_pad = pl.pallas_call(
        _dist_argmin_body,
        grid=(N_BLK,),
        in_specs=[
            pl.BlockSpec((BLK, DIM), lambda i: (i, 0)),
            pl.BlockSpec((N_EMB, DIM), lambda i: (0, 0)),
            pl.BlockSpec((BLK, 1), lambda i: (i, 0)),
            pl.BlockSpec((1, N_EMB), lambda i: (0, 0)),
        ],
        out_specs=[
            pl.BlockSpec((BLK // 128, 128), lambda i: (i, 0)),
            pl.BlockSpec((1, 1), lambda i: (0, 0)),
            pl.BlockSpec((N_EMB, PAD_D), lambda i: (0, 0)),
        ],
        out_shape=[
            jax.ShapeDtypeStruct((ROWS // 128, 128), jnp.int32),
            jax.ShapeDtypeStruct((1, 1), jnp.float32),
            jax.ShapeDtypeStruct((N_EMB, PAD_D), jnp.float32),
        ],
    )(z_flat, embedding_weight, zn, wn)

    idx_flat = idx2d.reshape(ROWS)
    idx_chunks = idx2d

    sc_gather = pl.kernel(
        _sc_gather_st_body,
        out_type=jax.ShapeDtypeStruct((ROWS, DIM), jnp.float32),
        scratch_types=[
            pltpu.VMEM((CHUNKS_PER_W, IDX_CHUNK), jnp.int32),
            pltpu.VMEM((B_PER_W, PAD_D), jnp.float32),
            pltpu.VMEM((B_PER_W, DIM), jnp.float32),
            pltpu.VMEM((B_PER_W, DIM), jnp.float32),
            pltpu.SemaphoreType.DMA,
        ],
        mesh=plsc.VectorSubcoreMesh(core_axis_name="c", subcore_axis_name="s"),
    )
    z_q_st = sc_gather(table_pad, idx_chunks, z_flat)

    return (z_q_st.reshape(z.shape), loss2d[0, 0], idx_flat.reshape(z.shape[:-1]))


# speedup vs baseline: 1.2603x; 1.0149x over previous
import jax
import jax.numpy as jnp
from jax import lax
from jax.experimental import pallas as pl
from jax.experimental.pallas import tpu as pltpu
from jax.experimental.pallas import tpu_sc as plsc

N_EMB = 8192
DIM = 32
ROWS = 8 * 1024
BLK = 1024
N_BLK = ROWS // BLK

N_WORKERS = 32
B_PER_W = ROWS // N_WORKERS


CHUNK = 2048
N_CHUNK = N_EMB // CHUNK

IDX_CHUNK = 128
CHUNKS_PER_W = B_PER_W // IDX_CHUNK
PAD_D = 128


def _dist_argmin_body(z_ref, w_ref, zn_ref, wn_ref, idx_ref, sse_ref, wpad_ref):
    zb = z_ref[...].astype(jnp.bfloat16)
    w = w_ref[...]
    a = zn_ref[...]
    b = wn_ref[...]
    m2 = lax.dot_general(zb, w * (-2.0), (((1,), (1,)), ((), ())),
                         preferred_element_type=jnp.float32)
    d = (a + b) + m2

    col = lax.broadcasted_iota(jnp.int32, (BLK, CHUNK), 1).astype(jnp.float32)
    acc_v = None
    for k in range(N_CHUNK):
        dk = d[:, k * CHUNK:(k + 1) * CHUNK]
        cmin = jnp.min(dk, axis=1)
        ci = jnp.min(jnp.where(dk == cmin[:, None], col, float(N_EMB)),
                     axis=1) + float(k * CHUNK)
        if acc_v is None:
            acc_v = cmin.astype(jnp.bfloat16).astype(jnp.float32)
            acc_i = ci
            acc_e = cmin
        else:
            better = cmin < acc_v
            acc_v = jnp.where(better, cmin.astype(jnp.bfloat16).astype(jnp.float32), acc_v)
            acc_i = jnp.where(better, ci, acc_i)
            acc_e = jnp.where(better, cmin, acc_e)
    idx_ref[...] = acc_i.astype(jnp.int32).reshape(BLK // 128, 128)

    @pl.when(pl.program_id(0) == 0)
    def _():
        sse_ref[...] = jnp.zeros_like(sse_ref)

    ws = w_ref[pl.ds(pl.program_id(0) * (N_EMB // N_BLK), N_EMB // N_BLK), :]
    wpad_ref[...] = jnp.pad(ws, ((0, 0), (0, PAD_D - DIM)))

    sse_ref[...] += jnp.sum(acc_e) * (1.25 / float(ROWS * DIM))


def _sc_gather_st_body(table_hbm, idx_hbm, z_hbm, out_hbm,
                       idx_v, rows_v, z_v, st_v, sem):
    wid = lax.axis_index("s") * 2 + lax.axis_index("c")
    base = wid * B_PER_W
    pltpu.sync_copy(idx_hbm.at[pl.ds(wid * CHUNKS_PER_W, CHUNKS_PER_W)], idx_v)
    copies = [
        pltpu.async_copy(table_hbm.at[idx_v.at[j]],
                         rows_v.at[pl.ds(j * IDX_CHUNK, IDX_CHUNK)], sem)
        for j in range(CHUNKS_PER_W)
    ]
    pltpu.sync_copy(z_hbm.at[pl.ds(base, B_PER_W)], z_v)
    for c in copies:
        c.wait()

    def body(r4, carry):
        for u in range(4):
            r = r4 * 4 + u
            for c in (0, 16):
                v = rows_v[r, pl.ds(c, 16)]
                zz = z_v[r, pl.ds(c, 16)]
                st_v[r, pl.ds(c, 16)] = zz + (v - zz)
        return carry

    lax.fori_loop(0, B_PER_W // 4, body, 0)
    pltpu.sync_copy(st_v, out_hbm.at[pl.ds(base, B_PER_W)])


def kernel(z, embedding_weight):
    z_flat = z.reshape(ROWS, DIM)
    zn = jnp.sum(z_flat ** 2, axis=1, keepdims=True)
    wn = jnp.sum(embedding_weight ** 2, axis=1).reshape(1, N_EMB)

    idx2d, loss2d, table_pad = pl.pallas_call(
        _dist_argmin_body,
        grid=(N_BLK,),
        in_specs=[
            pl.BlockSpec((BLK, DIM), lambda i: (i, 0)),
            pl.BlockSpec((N_EMB, DIM), lambda i: (0, 0)),
            pl.BlockSpec((BLK, 1), lambda i: (i, 0)),
            pl.BlockSpec((1, N_EMB), lambda i: (0, 0)),
        ],
        out_specs=[
            pl.BlockSpec((BLK // 128, 128), lambda i: (i, 0)),
            pl.BlockSpec((1, 1), lambda i: (0, 0)),
            pl.BlockSpec((N_EMB // N_BLK, PAD_D), lambda i: (i, 0)),
        ],
        out_shape=[
            jax.ShapeDtypeStruct((ROWS // 128, 128), jnp.int32),
            jax.ShapeDtypeStruct((1, 1), jnp.float32),
            jax.ShapeDtypeStruct((N_EMB, PAD_D), jnp.float32),
        ],
    )(z_flat, embedding_weight, zn, wn)

    idx_flat = idx2d.reshape(ROWS)
    idx_chunks = idx2d

    sc_gather = pl.kernel(
        _sc_gather_st_body,
        out_type=jax.ShapeDtypeStruct((ROWS, DIM), jnp.float32),
        scratch_types=[
            pltpu.VMEM((CHUNKS_PER_W, IDX_CHUNK), jnp.int32),
            pltpu.VMEM((B_PER_W, PAD_D), jnp.float32),
            pltpu.VMEM((B_PER_W, DIM), jnp.float32),
            pltpu.VMEM((B_PER_W, DIM), jnp.float32),
            pltpu.SemaphoreType.DMA,
        ],
        mesh=plsc.VectorSubcoreMesh(core_axis_name="c", subcore_axis_name="s"),
    )
    z_q_st = sc_gather(table_pad, idx_chunks, z_flat)

    return (z_q_st.reshape(z.shape), loss2d[0, 0], idx_flat.reshape(z.shape[:-1]))
